# baseline (speedup 1.0000x reference)
import jax
import jax.numpy as jnp
from jax.experimental import pallas as pl

RES = 64
PAD = 0.1
DIM = 32


def _normalize_coordinate(p, padding):
    p_nor = p / (1.0 + padding + 1e-3) + 0.5
    return jnp.clip(p_nor, 0.0, 1.0 - 1e-6)


def _xyz_to_triplane_indices(xyz, r, padding):
    planes = [(0, 1), (0, 2), (1, 2)]
    idxs = []
    for (a, b2) in planes:
        p = _normalize_coordinate(jnp.stack([xyz[..., a], xyz[..., b2]], axis=-1), padding)
        ij = jnp.clip((p * r).astype(jnp.int32), 0, r - 1)
        idxs.append(ij[..., 0] + r * ij[..., 1])
    return jnp.stack(idxs, axis=1)


def _feature_to_triplane(c, tri_idx, res, reduction='mean'):
    b, n, d = c.shape
    offsets = (jnp.arange(b, dtype=jnp.int32) * res * res)[:, None]
    cf = c.reshape(b * n, d)
    planes = []
    for p in range(3):
        flat = (tri_idx[:, p, :] + offsets).reshape(-1)
        if reduction == 'max':
            grid = jnp.full((b * res * res, d), -jnp.inf, c.dtype).at[flat].max(cf)
            grid = jnp.where(jnp.isinf(grid), 0.0, grid)
        else:
            s = jnp.zeros((b * res * res, d), c.dtype).at[flat].add(cf)
            cnt = jnp.zeros((b * res * res,), c.dtype).at[flat].add(1.0)
            grid = s / jnp.clip(cnt, 1.0)[:, None]
        planes.append(grid.reshape(b, res * res, d))
    return jnp.stack(planes, axis=1)


def _triplane_to_point(tri_feat, tri_idx):
    outs = []
    for p in range(3):
        outs.append(jnp.take_along_axis(tri_feat[:, p], tri_idx[:, p][..., None], axis=1))
    return outs


def _residual_mlp(x, w):
    h = jax.nn.relu(x)
    h = h @ w['fc0_W'] + w['fc0_b']
    h = jax.nn.relu(h)
    h = h @ w['fc1_W'] + w['fc1_b']
    xs = x @ w['sc_W'] + w['sc_b']
    return xs + h


def kernel(x, params):
    xyz = x[..., :3]
    tri_idx = _xyz_to_triplane_indices(xyz, RES, PAD)
    h = x @ params['stem_W'] + params['stem_b']
    h = _residual_mlp(h, params['blocks'][0])
    for blk in params['blocks'][1:]:
        tri_feat = _feature_to_triplane(h, tri_idx, RES, reduction='max')
        tri_c = _triplane_to_point(tri_feat, tri_idx)
        pooled = tri_c[0] + tri_c[1] + tri_c[2]
        h = jnp.concatenate([h, pooled], axis=-1)
        h = _residual_mlp(h, blk)
    c = h @ params['fc_c_W'] + params['fc_c_b']
    tri_feat = _feature_to_triplane(c, tri_idx, RES, reduction='mean')
    return (xyz, c, tri_feat)


# speedup vs baseline: 5.9788x; 5.9788x over previous
import functools

import jax
import jax.numpy as jnp
from jax import lax
from jax.experimental import pallas as pl
from jax.experimental.pallas import tpu as pltpu
from jax.experimental.pallas import tpu_sc as plsc

RES = 64
PAD = 0.1
DIM = 32
NPL = 3
CELLS = RES * RES
QW = 8
GRIDW = NPL * CELLS * QW
K = 256
NB = 2048


def _cells8(x):
    planes = [(0, 1), (0, 2), (1, 2)]
    cs = []
    for (a, b2) in planes:
        p = jnp.stack([x[..., a], x[..., b2]], axis=-1)
        p = p / (1.0 + PAD + 1e-3) + 0.5
        p = jnp.clip(p, 0.0, 1.0 - 1e-6)
        ij = jnp.clip((p * RES).astype(jnp.int32), 0, RES - 1)
        cs.append(ij[..., 0] + RES * ij[..., 1])
    c = jnp.stack(cs, axis=1)
    return (c * 8).reshape(x.shape[0], -1)


def _stem0_body(x_ref, sw, sb, w0, b0, w1, b1, ws, bs, out_ref):
    x = x_ref[0]
    t = jnp.dot(x, sw[...], preferred_element_type=jnp.float32) + sb[...]
    net = jnp.maximum(t, 0.0)
    net = jnp.dot(net, w0[...], preferred_element_type=jnp.float32) + b0[...]
    net = jnp.maximum(net, 0.0)
    net = jnp.dot(net, w1[...], preferred_element_type=jnp.float32) + b1[...]
    sc = jnp.dot(t, ws[...], preferred_element_type=jnp.float32) + bs[...]
    out_ref[0] = sc + net


def _stem0_tc(x, params):
    B, N, _ = x.shape
    p = params
    b0 = p['blocks'][0]
    w_args = (p['stem_W'], p['stem_b'].reshape(1, -1),
              b0['fc0_W'], b0['fc0_b'].reshape(1, -1),
              b0['fc1_W'], b0['fc1_b'].reshape(1, -1),
              b0['sc_W'], b0['sc_b'].reshape(1, -1))
    w_specs = [pl.BlockSpec(w.shape, lambda bb, i: (0, 0)) for w in w_args]
    return pl.pallas_call(
        _stem0_body,
        grid=(B, N // NB),
        in_specs=[pl.BlockSpec((1, NB, 3), lambda bb, i: (bb, i, 0))] + w_specs,
        out_specs=pl.BlockSpec((1, NB, DIM), lambda bb, i: (bb, i, 0)),
        out_shape=jax.ShapeDtypeStruct((B, N, DIM), jnp.float32),
    )(x, *w_args)


def _round_body(h_ref, p0, p1, p2, p3, w0, b0, w1, b1, ws, bs, out_ref):
    x = jnp.concatenate(
        [h_ref[0], p0[0, 0], p1[0, 0], p2[0, 0], p3[0, 0]], axis=-1)
    net = jnp.maximum(x, 0.0)
    net = jnp.dot(net, w0[...], preferred_element_type=jnp.float32) + b0[...]
    net = jnp.maximum(net, 0.0)
    net = jnp.dot(net, w1[...], preferred_element_type=jnp.float32) + b1[...]
    sc = jnp.dot(x, ws[...], preferred_element_type=jnp.float32) + bs[...]
    out_ref[0] = sc + net


def _round_tc(h, pooled4, blk):
    B, N, _ = h.shape
    w_args = (blk['fc0_W'], blk['fc0_b'].reshape(1, -1),
              blk['fc1_W'], blk['fc1_b'].reshape(1, -1),
              blk['sc_W'], blk['sc_b'].reshape(1, -1))
    w_specs = [pl.BlockSpec(w.shape, lambda bb, i: (0, 0)) for w in w_args]
    q_specs = [
        pl.BlockSpec((1, 1, NB, QW),
                     functools.partial(lambda bb, i, q: (bb, q, i, 0), q=q))
        for q in range(4)
    ]
    return pl.pallas_call(
        _round_body,
        grid=(B, N // NB),
        in_specs=[pl.BlockSpec((1, NB, DIM), lambda bb, i: (bb, i, 0))]
        + q_specs + w_specs,
        out_specs=pl.BlockSpec((1, NB, DIM), lambda bb, i: (bb, i, 0)),
        out_shape=jax.ShapeDtypeStruct((B, N, DIM), jnp.float32),
    )(h, pooled4, pooled4, pooled4, pooled4, *w_args)


def _fc_body(h_ref, w, b, out_ref):
    out_ref[0] = (jnp.dot(h_ref[0], w[...], preferred_element_type=jnp.float32)
                  + b[...])


def _fc_tc(h, w, b):
    B, N, _ = h.shape
    w_args = (w, b.reshape(1, -1))
    w_specs = [pl.BlockSpec(a.shape, lambda bb, i: (0, 0)) for a in w_args]
    return pl.pallas_call(
        _fc_body,
        grid=(B, N // NB),
        in_specs=[pl.BlockSpec((1, NB, DIM), lambda bb, i: (bb, i, 0))] + w_specs,
        out_specs=pl.BlockSpec((1, NB, DIM), lambda bb, i: (bb, i, 0)),
        out_shape=jax.ShapeDtypeStruct((B, N, DIM), jnp.float32),
    )(h, *w_args)


def _divide_body(s0, s1, s2, s3, cnt_ref, out_ref):
    s = jnp.concatenate(
        [s0[0, 0, 0], s1[0, 0, 0], s2[0, 0, 0], s3[0, 0, 0]], axis=-1)
    cnt = jnp.maximum(cnt_ref[0, 0], 1.0)
    out_ref[0, 0] = s / cnt[:, None]


def _cnt_spec():
    return pl.BlockSpec((1, 1, CELLS), lambda bb, p: (bb * NPL + p, 0, 0))


def _divide_tc(sums5, cnt3):
    B = sums5.shape[0]
    q_specs = [
        pl.BlockSpec((1, 1, 1, CELLS, QW),
                     functools.partial(lambda bb, p, q: (bb, q, p, 0, 0), q=q))
        for q in range(4)
    ]
    return pl.pallas_call(
        _divide_body,
        grid=(B, NPL),
        in_specs=q_specs + [_cnt_spec()],
        out_specs=pl.BlockSpec((1, 1, CELLS, DIM), lambda bb, p: (bb, p, 0, 0)),
        out_shape=jax.ShapeDtypeStruct((B, NPL, CELLS, DIM), jnp.float32),
    )(sums5, sums5, sums5, sums5, cnt3)


def _dg(x, idx):
    return lax.gather(
        x, idx[:, None],
        lax.GatherDimensionNumbers(
            offset_dims=(), collapsed_slice_dims=(0,), start_index_map=(0,)),
        slice_sizes=(1,),
        mode=lax.GatherScatterMode.PROMISE_IN_BOUNDS)


def _io16():
    return lax.iota(jnp.int32, 16)


def _sc_round_fn(B, N):
    mesh = plsc.VectorSubcoreMesh(core_axis_name="c", subcore_axis_name="s")
    nch = N // K

    @functools.partial(
        pl.kernel, mesh=mesh,
        out_type=jax.ShapeDtypeStruct((B * 4 * N * QW,), jnp.float32),
        scratch_types=[
            pltpu.VMEM((GRIDW,), jnp.float32),
            pltpu.VMEM((NPL * K,), jnp.int32), pltpu.VMEM((NPL * K,), jnp.int32),
            pltpu.VMEM((K * DIM,), jnp.float32), pltpu.VMEM((K * DIM,), jnp.float32),
            pltpu.VMEM((K * QW,), jnp.float32), pltpu.VMEM((K * QW,), jnp.float32),
            pltpu.SemaphoreType.DMA, pltpu.SemaphoreType.DMA,
            pltpu.SemaphoreType.DMA, pltpu.SemaphoreType.DMA,
        ],
        compiler_params=pltpu.CompilerParams(needs_layout_passes=False),
    )
    def k(cells8, hflat, pooled, grid_v, ix0, ix1, hv0, hv1, ov0, ov1,
          si0, si1, so0, so1):
        cid = lax.axis_index("c")
        sid = lax.axis_index("s")
        b = cid * (B // 2) + sid // 4
        q = sid % 4
        ixv = (ix0, ix1)
        hv = (hv0, hv1)
        ov = (ov0, ov1)
        sin = (si0, si1)
        sout = (so0, so1)
        io = _io16()
        io8 = io & 7
        m8 = io < 8
        pbase = [jnp.full((16,), p * CELLS * QW, jnp.int32) + io8
                 for p in range(3)]
        q8 = q * QW
        neg = jnp.full((16,), -jnp.inf, jnp.float32)

        def idx_copies(g, s, want_h):
            off = pl.multiple_of(g * K, K)
            cps = [pltpu.make_async_copy(
                cells8.at[b, pl.ds(p * N + off, K)],
                ixv[s].at[pl.ds(p * K, K)], sin[s])
                for p in range(3)]
            if want_h:
                hoff = pl.multiple_of(g * (K * DIM), K * DIM)
                cps.append(pltpu.make_async_copy(
                    hflat.at[b, pl.ds(hoff, K * DIM)], hv[s], sin[s]))
            return cps

        def issue(g, s, want_h):
            for c in idx_copies(g, s, want_h):
                c.start()

        def drain(g, s, want_h):
            for c in idx_copies(g, s, want_h):
                c.wait()

        def out_copy(g, s):
            base = (b * 4 + q) * (N * QW)
            off = pl.multiple_of(g * (K * QW), K * QW)
            return pltpu.make_async_copy(
                ov[s], pooled.at[pl.ds(base + off, K * QW)], sout[s])

        @pl.loop(0, GRIDW // 16)
        def _(i):
            grid_v[pl.ds(pl.multiple_of(i * 16, 16), 16)] = neg

        def scatter_chunk(s):
            @pl.loop(0, K // 16)
            def _(g16):
                goff = pl.multiple_of(g16 * 16, 16)
                ios = [ixv[s][pl.ds(p * K + goff, 16)] for p in range(3)]
                for j in range(16):
                    jv = jnp.full((16,), j, jnp.int32)
                    fidx = (goff + j) * DIM + q8 + io8
                    fj = plsc.load_gather(hv[s], [fidx])
                    for p in range(3):
                        off = _dg(ios[p], jv) + pbase[p]
                        g0 = plsc.load_gather(grid_v, [off], mask=m8)
                        plsc.store_scatter(grid_v, [off],
                                           jnp.maximum(g0, fj), mask=m8)

        issue(0, 0, True)
        issue(1, 1, True)

        @pl.loop(0, nch // 2)
        def _(gg):
            for s in range(2):
                g = gg * 2 + s
                drain(g, s, True)

                @pl.when(g + 2 < nch)
                def _():
                    issue(g + 2, s, True)

                scatter_chunk(s)

        issue(0, 0, False)
        issue(1, 1, False)

        @pl.loop(0, nch // 2)
        def _(gg):
            for s in range(2):
                g = gg * 2 + s
                drain(g, s, False)

                @pl.when(g + 2 < nch)
                def _():
                    issue(g + 2, s, False)

                @pl.when(g >= 2)
                def _():
                    out_copy(g - 2, s).wait()

                @pl.loop(0, K // 16)
                def _(g16):
                    goff = pl.multiple_of(g16 * 16, 16)
                    ios = [ixv[s][pl.ds(p * K + goff, 16)] for p in range(3)]
                    for j in range(16):
                        jv = jnp.full((16,), j, jnp.int32)
                        acc = plsc.load_gather(
                            grid_v, [_dg(ios[0], jv) + pbase[0]], mask=m8)
                        for p in (1, 2):
                            acc = acc + plsc.load_gather(
                                grid_v, [_dg(ios[p], jv) + pbase[p]], mask=m8)
                        plsc.store_scatter(
                            ov[s], [jnp.full((16,), (goff + j) * QW, jnp.int32)
                                    + io8],
                            acc, mask=m8)

                out_copy(g, s).start()

        out_copy(nch - 2, 0).wait()
        out_copy(nch - 1, 1).wait()

    return k


def _sc_mean_fn(B, N):
    mesh = plsc.VectorSubcoreMesh(core_axis_name="c", subcore_axis_name="s")
    nch = N // K

    @functools.partial(
        pl.kernel, mesh=mesh,
        out_type=(jax.ShapeDtypeStruct((B * 4 * GRIDW,), jnp.float32),
                  jax.ShapeDtypeStruct((B * NPL * CELLS,), jnp.float32)),
        scratch_types=[
            pltpu.VMEM((GRIDW,), jnp.float32),
            pltpu.VMEM((CELLS,), jnp.float32),
            pltpu.VMEM((NPL * K,), jnp.int32), pltpu.VMEM((NPL * K,), jnp.int32),
            pltpu.VMEM((K * DIM,), jnp.float32), pltpu.VMEM((K * DIM,), jnp.float32),
            pltpu.SemaphoreType.DMA, pltpu.SemaphoreType.DMA,
        ],
        compiler_params=pltpu.CompilerParams(needs_layout_passes=False),
    )
    def k(cells8, cflat, sums, cnt, grid_v, cnt_v, ix0, ix1, hv0, hv1,
          si0, si1):
        cid = lax.axis_index("c")
        sid = lax.axis_index("s")
        b = cid * (B // 2) + sid // 4
        q = sid % 4
        ixv = (ix0, ix1)
        hv = (hv0, hv1)
        sin = (si0, si1)
        io = _io16()
        io8 = io & 7
        m8 = io < 8
        m1 = io < 1
        ones = jnp.full((16,), 1.0, jnp.float32)
        zeros = jnp.zeros((16,), jnp.float32)
        pbase = [jnp.full((16,), p * CELLS * QW, jnp.int32) + io8
                 for p in range(3)]
        q8 = q * QW

        def idx_copies(g, s):
            off = pl.multiple_of(g * K, K)
            cps = [pltpu.make_async_copy(
                cells8.at[b, pl.ds(p * N + off, K)],
                ixv[s].at[pl.ds(p * K, K)], sin[s])
                for p in range(3)]
            hoff = pl.multiple_of(g * (K * DIM), K * DIM)
            cps.append(pltpu.make_async_copy(
                cflat.at[b, pl.ds(hoff, K * DIM)], hv[s], sin[s]))
            return cps

        @pl.loop(0, GRIDW // 16)
        def _(i):
            grid_v[pl.ds(pl.multiple_of(i * 16, 16), 16)] = zeros

        @pl.loop(0, CELLS // 16)
        def _(i):
            cnt_v[pl.ds(pl.multiple_of(i * 16, 16), 16)] = zeros

        for c in idx_copies(0, 0):
            c.start()
        for c in idx_copies(1, 1):
            c.start()

        @pl.loop(0, nch // 2)
        def _(gg):
            for s in range(2):
                g = gg * 2 + s
                for c in idx_copies(g, s):
                    c.wait()

                @pl.when(g + 2 < nch)
                def _():
                    for c in idx_copies(g + 2, s):
                        c.start()

                @pl.loop(0, K // 16)
                def _(g16):
                    goff = pl.multiple_of(g16 * 16, 16)
                    ios = [ixv[s][pl.ds(p * K + goff, 16)] for p in range(3)]
                    for j in range(16):
                        jv = jnp.full((16,), j, jnp.int32)
                        fidx = (goff + j) * DIM + q8 + io8
                        fj = plsc.load_gather(hv[s], [fidx])
                        bps = [_dg(ios[p], jv) for p in range(3)]
                        for p in range(3):
                            plsc.addupdate_scatter(
                                grid_v, [bps[p] + pbase[p]], fj, mask=m8)

                        @pl.when(q < 3)
                        def _():
                            csel = jnp.where(
                                q == 0, bps[0],
                                jnp.where(q == 1, bps[1], bps[2]))
                            plsc.addupdate_scatter(
                                cnt_v, [lax.shift_right_logical(csel, 3)],
                                ones, mask=m1)

        pltpu.sync_copy(grid_v, sums.at[pl.ds((b * 4 + q) * GRIDW, GRIDW)])

        @pl.when(q < 3)
        def _():
            pltpu.sync_copy(cnt_v, cnt.at[pl.ds((b * NPL + q) * CELLS, CELLS)])

    return k


def kernel(x, params):
    B, N, _ = x.shape
    cells8 = _cells8(x)

    h = _stem0_tc(x, params)

    sc_round = _sc_round_fn(B, N)
    for blk in params['blocks'][1:]:
        pooled = sc_round(cells8, h.reshape(B, N * DIM))
        h = _round_tc(h, pooled.reshape(B, 4, N, QW), blk)

    c = _fc_tc(h, params['fc_c_W'], params['fc_c_b'])

    sums, cnt = _sc_mean_fn(B, N)(cells8, c.reshape(B, N * DIM))
    tri_feat = _divide_tc(sums.reshape(B, 4, NPL, CELLS, QW),
                          cnt.reshape(B * NPL, 1, CELLS))

    return (x[..., :3], c, tri_feat)
